# baseline (speedup 1.0000x reference)
import functools

import jax
import jax.numpy as jnp
from jax import lax
from jax.experimental import pallas as pl
from jax.experimental.pallas import tpu as pltpu
from jax.experimental.pallas import tpu_sc as plsc

_NUM_BUCKETS = 1000000
_HASH_DIM = 64
_MODEL_DIM = 1024
_MULT = 92821

_NC = 2
_NS = 16
_NW = _NC * _NS
_L = 16

_TOK = 32768
_CHUNK = _TOK // _NW
_GBLK = 128
_NG = _CHUNK // _GBLK


def _sc_gather(ids_hbm, prev_hbm, table_hbm, emb_hbm,
               ids_v, prev_v, h_v, rows_v, sem):
    wid = lax.axis_index("s") * _NC + lax.axis_index("c")
    base = wid * _CHUNK
    pltpu.sync_copy(ids_hbm.at[pl.ds(base, _CHUNK)], ids_v)
    pltpu.sync_copy(prev_hbm.at[pl.ds(base, _CHUNK)], prev_v)
    for i in range(_CHUNK // _L):
        a = prev_v[pl.ds(i * _L, _L)]
        b = ids_v[pl.ds(i * _L, _L)]
        t = a * _MULT + b
        r = lax.rem(t, _NUM_BUCKETS)
        h = jnp.where(r < 0, r + _NUM_BUCKETS, r)
        h_v[i // (_GBLK // _L), pl.ds((i % (_GBLK // _L)) * _L, _L)] = h
    descs = []
    for j in range(_NG):
        descs.append(
            pltpu.async_copy(table_hbm.at[h_v.at[j]],
                             rows_v.at[pl.ds(j * _GBLK, _GBLK)], sem))
    for d in descs:
        d.wait()
    pltpu.sync_copy(rows_v, emb_hbm.at[pl.ds(base, _CHUNK)])


_gather_call = functools.partial(
    pl.kernel,
    out_type=jax.ShapeDtypeStruct((_TOK, _HASH_DIM), jnp.bfloat16),
    mesh=plsc.VectorSubcoreMesh(
        core_axis_name="c", subcore_axis_name="s",
        num_cores=_NC, num_subcores=_NS),
    scratch_types=[
        pltpu.VMEM((_CHUNK,), jnp.int32),
        pltpu.VMEM((_CHUNK,), jnp.int32),
        pltpu.VMEM((_NG, _GBLK), jnp.int32),
        pltpu.VMEM((_CHUNK, _HASH_DIM), jnp.bfloat16),
        pltpu.SemaphoreType.DMA,
    ],
    compiler_params=pltpu.CompilerParams(use_tc_tiling_on_sc=False),
)(_sc_gather)


def _mm_body(emb_ref, wt_ref, o_ref):
    o_ref[...] = lax.dot_general(
        emb_ref[...], wt_ref[...], (((1,), (0,)), ((), ())),
        preferred_element_type=jnp.float32)


def kernel(input_ids, table, W):
    ids = input_ids.astype(jnp.int32)
    bsz, seqlen = ids.shape
    prev = jnp.concatenate(
        [jnp.zeros((bsz, 1), dtype=ids.dtype), ids[:, :-1]], axis=1)
    ids_f = ids.reshape(-1)
    prev_f = prev.reshape(-1)

    emb = _gather_call(ids_f, prev_f, table.astype(jnp.bfloat16))

    blk = 512
    out = pl.pallas_call(
        _mm_body,
        grid=(_TOK // blk,),
        in_specs=[
            pl.BlockSpec((blk, _HASH_DIM), lambda i: (i, 0)),
            pl.BlockSpec((_HASH_DIM, _MODEL_DIM), lambda i: (0, 0)),
        ],
        out_specs=pl.BlockSpec((blk, _MODEL_DIM), lambda i: (i, 0)),
        out_shape=jax.ShapeDtypeStruct((_TOK, _MODEL_DIM), jnp.float32),
    )(emb, W.T.astype(jnp.bfloat16))
    return out.reshape(bsz, seqlen, _MODEL_DIM)


# speedup vs baseline: 3.0019x; 3.0019x over previous
import functools

import jax
import jax.numpy as jnp
from jax import lax
from jax.experimental import pallas as pl
from jax.experimental.pallas import tpu as pltpu
from jax.experimental.pallas import tpu_sc as plsc

_NUM_BUCKETS = 1000000
_HASH_DIM = 64
_MODEL_DIM = 1024
_MULT = 92821

_NC = 2
_NS = 16
_L = 16

_TOK = 32768
_TPT = _TOK // _NS
_FPC = _HASH_DIM // _NC
_GBLK = 128
_NG = _TPT // _GBLK


_IDC = 256


def _sc_gather(ids_hbm, prev_hbm, tabT_hbm, embT_hbm,
               ids_v, prev_v, h_v, stage_v, buf0, sem0, gsem):
    cid = lax.axis_index("c")
    sid = lax.axis_index("s")
    base = sid * _TPT
    f0 = cid * _FPC
    def _hash_body(c, carry):
        pltpu.sync_copy(ids_hbm.at[pl.ds(base + c * _IDC, _IDC)], ids_v)
        pltpu.sync_copy(prev_hbm.at[pl.ds(base + c * _IDC, _IDC)], prev_v)
        for k in range(_IDC // _L):
            a = prev_v[pl.ds(k * _L, _L)]
            b = ids_v[pl.ds(k * _L, _L)]
            t = a * _MULT + b
            r = lax.rem(t, _NUM_BUCKETS)
            h = jnp.where(r < 0, r + _NUM_BUCKETS, r)
            h_v[c * (_IDC // _GBLK) + k // (_GBLK // _L),
                pl.ds((k % (_GBLK // _L)) * _L, _L)] = h
        return carry

    lax.fori_loop(0, _TPT // _IDC, _hash_body, 0)

    def _feature_body(f, carry):
        @pl.when(sid == 0)
        def _():
            pltpu.async_copy(tabT_hbm.at[f0 + f], buf0, sem0).wait()

        plsc.subcore_barrier()
        for half in range(2):
            gds = []
            for j in range(_NG // 2):
                jj = half * (_NG // 2) + j
                gds.append(pltpu.async_copy(
                    buf0.at[h_v.at[jj]],
                    stage_v.at[pl.ds(j * _GBLK, _GBLK)], gsem))
            for d in gds:
                d.wait()
            pltpu.sync_copy(
                stage_v,
                embT_hbm.at[f0 + f,
                            pl.ds(base + half * (_TPT // 2), _TPT // 2)])
        plsc.subcore_barrier()
        return carry

    lax.fori_loop(0, _FPC, _feature_body, 0)


_gather_call = functools.partial(
    pl.kernel,
    out_type=jax.ShapeDtypeStruct((_HASH_DIM, _TOK), jnp.float32),
    mesh=plsc.VectorSubcoreMesh(
        core_axis_name="c", subcore_axis_name="s",
        num_cores=_NC, num_subcores=_NS),
    scratch_types=[
        pltpu.VMEM((_IDC,), jnp.int32),
        pltpu.VMEM((_IDC,), jnp.int32),
        pltpu.VMEM((_NG, _GBLK), jnp.int32),
        pltpu.VMEM((_TPT // 2,), jnp.float32),
        pltpu.VMEM_SHARED((_NUM_BUCKETS,), jnp.float32),
        pltpu.SemaphoreType.DMA,
        pltpu.SemaphoreType.DMA,
    ],
)(_sc_gather)


def _mm_body(embT_ref, w_ref, o_ref):
    o_ref[...] = lax.dot_general(
        embT_ref[...], w_ref[...], (((0,), (1,)), ((), ())),
        preferred_element_type=jnp.float32)


def kernel(input_ids, table, W):
    ids = input_ids.astype(jnp.int32)
    bsz, seqlen = ids.shape
    prev = jnp.concatenate(
        [jnp.zeros((bsz, 1), dtype=ids.dtype), ids[:, :-1]], axis=1)
    ids_f = ids.reshape(-1)
    prev_f = prev.reshape(-1)

    embT = _gather_call(ids_f, prev_f, table.T)

    blk = 512
    out = pl.pallas_call(
        _mm_body,
        grid=(_TOK // blk,),
        in_specs=[
            pl.BlockSpec((_HASH_DIM, blk), lambda i: (0, i)),
            pl.BlockSpec((_MODEL_DIM, _HASH_DIM), lambda i: (0, 0)),
        ],
        out_specs=pl.BlockSpec((blk, _MODEL_DIM), lambda i: (i, 0)),
        out_shape=jax.ShapeDtypeStruct((_TOK, _MODEL_DIM), jnp.float32),
    )(embT, W)
    return out.reshape(bsz, seqlen, _MODEL_DIM)


# speedup vs baseline: 3.3044x; 1.1008x over previous
import functools

import jax
import jax.numpy as jnp
from jax import lax
from jax.experimental import pallas as pl
from jax.experimental.pallas import tpu as pltpu
from jax.experimental.pallas import tpu_sc as plsc

_NUM_BUCKETS = 1000000
_HASH_DIM = 64
_MODEL_DIM = 1024
_MULT = 92821

_NC = 2
_NS = 16
_L = 16

_TOK = 32768
_TPT = _TOK // _NS
_FPC = _HASH_DIM // _NC
_GBLK = 128
_NG = _TPT // _GBLK


_IDC = 256


def _sc_gather(ids_hbm, prev_hbm, tabT_hbm, embT_hbm,
               ids_v, prev_v, h_v, stage_v, buf0, buf1, sem0, sem1, gsem):
    cid = lax.axis_index("c")
    sid = lax.axis_index("s")
    base = sid * _TPT
    f0 = cid * _FPC
    def _hash_body(c, carry):
        pltpu.sync_copy(ids_hbm.at[pl.ds(base + c * _IDC, _IDC)], ids_v)
        pltpu.sync_copy(prev_hbm.at[pl.ds(base + c * _IDC, _IDC)], prev_v)
        for k in range(_IDC // _L):
            a = prev_v[pl.ds(k * _L, _L)]
            b = ids_v[pl.ds(k * _L, _L)]
            t = a * _MULT + b
            r = lax.rem(t, _NUM_BUCKETS)
            h = jnp.where(r < 0, r + _NUM_BUCKETS, r)
            h_v[c * (_IDC // _GBLK) + k // (_GBLK // _L),
                pl.ds((k % (_GBLK // _L)) * _L, _L)] = h
        return carry

    lax.fori_loop(0, _TPT // _IDC, _hash_body, 0)

    _NST = _TPT // 512

    def _gather_row(buf, f):
        for q in range(_NST):
            gds = []
            for j in range(4):
                jj = q * 4 + j
                gds.append(pltpu.async_copy(
                    buf.at[h_v.at[jj]],
                    stage_v.at[pl.ds(j * _GBLK, _GBLK)], gsem))
            for d in gds:
                d.wait()
            pltpu.sync_copy(
                stage_v,
                embT_hbm.at[f, pl.ds(base + q * 512, 512)])

    @pl.when(sid == 0)
    def _():
        pltpu.async_copy(tabT_hbm.at[f0], buf0, sem0).wait()

    plsc.subcore_barrier()

    def _feature_body(i, carry):
        fa = f0 + 2 * i
        d1 = pltpu.make_async_copy(tabT_hbm.at[fa + 1], buf1, sem1)

        @pl.when(sid == 0)
        def _():
            d1.start()

        _gather_row(buf0, fa)

        @pl.when(sid == 0)
        def _():
            d1.wait()

        plsc.subcore_barrier()
        more = jnp.logical_and(sid == 0, i < _FPC // 2 - 1)
        d0 = pltpu.make_async_copy(
            tabT_hbm.at[jnp.minimum(fa + 2, _HASH_DIM - 1)], buf0, sem0)

        @pl.when(more)
        def _():
            d0.start()

        _gather_row(buf1, fa + 1)

        @pl.when(more)
        def _():
            d0.wait()

        plsc.subcore_barrier()
        return carry

    lax.fori_loop(0, _FPC // 2, _feature_body, 0)


_gather_call = functools.partial(
    pl.kernel,
    out_type=jax.ShapeDtypeStruct((_HASH_DIM, _TOK), jnp.float32),
    mesh=plsc.VectorSubcoreMesh(
        core_axis_name="c", subcore_axis_name="s",
        num_cores=_NC, num_subcores=_NS),
    scratch_types=[
        pltpu.VMEM((_IDC,), jnp.int32),
        pltpu.VMEM((_IDC,), jnp.int32),
        pltpu.VMEM((_NG, _GBLK), jnp.int32),
        pltpu.VMEM((512,), jnp.float32),
        pltpu.VMEM_SHARED((_NUM_BUCKETS,), jnp.float32),
        pltpu.VMEM_SHARED((_NUM_BUCKETS,), jnp.float32),
        pltpu.SemaphoreType.DMA,
        pltpu.SemaphoreType.DMA,
        pltpu.SemaphoreType.DMA,
    ],
)(_sc_gather)


def _mm_body(embT_ref, w_ref, o_ref):
    o_ref[...] = lax.dot_general(
        embT_ref[...], w_ref[...], (((0,), (1,)), ((), ())),
        preferred_element_type=jnp.float32)


def kernel(input_ids, table, W):
    ids = input_ids.astype(jnp.int32)
    bsz, seqlen = ids.shape
    prev = jnp.concatenate(
        [jnp.zeros((bsz, 1), dtype=ids.dtype), ids[:, :-1]], axis=1)
    ids_f = ids.reshape(-1)
    prev_f = prev.reshape(-1)

    embT = _gather_call(ids_f, prev_f, table.T)

    blk = 512
    out = pl.pallas_call(
        _mm_body,
        grid=(_TOK // blk,),
        in_specs=[
            pl.BlockSpec((_HASH_DIM, blk), lambda i: (0, i)),
            pl.BlockSpec((_MODEL_DIM, _HASH_DIM), lambda i: (0, 0)),
        ],
        out_specs=pl.BlockSpec((blk, _MODEL_DIM), lambda i: (i, 0)),
        out_shape=jax.ShapeDtypeStruct((_TOK, _MODEL_DIM), jnp.float32),
    )(embT, W)
    return out.reshape(bsz, seqlen, _MODEL_DIM)


# speedup vs baseline: 3.4875x; 1.0554x over previous
import functools

import jax
import jax.numpy as jnp
from jax import lax
from jax.experimental import pallas as pl
from jax.experimental.pallas import tpu as pltpu
from jax.experimental.pallas import tpu_sc as plsc

_NUM_BUCKETS = 1000000
_HASH_DIM = 64
_MODEL_DIM = 1024
_MULT = 92821

_NC = 2
_NS = 16
_L = 16

_TOK = 32768
_TPT = _TOK // _NS
_FPC = _HASH_DIM // _NC
_GBLK = 128
_NG = _TPT // _GBLK


_IDC = 256


def _sc_gather(ids_hbm, prev_hbm, tabT_hbm, embT_hbm,
               ids_v, prev_v, h_v, stage_v, buf0, buf1, sem0, sem1, gsem,
               wsem):
    cid = lax.axis_index("c")
    sid = lax.axis_index("s")
    base = sid * _TPT
    f0 = cid * _FPC
    def _hash_body(c, carry):
        pltpu.sync_copy(ids_hbm.at[pl.ds(base + c * _IDC, _IDC)], ids_v)
        pltpu.sync_copy(prev_hbm.at[pl.ds(base + c * _IDC, _IDC)], prev_v)
        for k in range(_IDC // _L):
            a = prev_v[pl.ds(k * _L, _L)]
            b = ids_v[pl.ds(k * _L, _L)]
            t = a * _MULT + b
            r = lax.rem(t, _NUM_BUCKETS)
            h = jnp.where(r < 0, r + _NUM_BUCKETS, r)
            h_v[c * (_IDC // _GBLK) + k // (_GBLK // _L),
                pl.ds((k % (_GBLK // _L)) * _L, _L)] = h
        return carry

    lax.fori_loop(0, _TPT // _IDC, _hash_body, 0)

    _NST = _TPT // 512

    def _gather_row(buf, f):
        wds = []
        for q in range(_NST):
            st = stage_v.at[pl.ds((q % 2) * 512, 512)]
            if q >= 2:
                wds[q - 2].wait()
            gds = []
            for j in range(4):
                jj = q * 4 + j
                gds.append(pltpu.async_copy(
                    buf.at[h_v.at[jj]],
                    st.at[pl.ds(j * _GBLK, _GBLK)], gsem))
            for d in gds:
                d.wait()
            w = pltpu.make_async_copy(
                st, embT_hbm.at[f, pl.ds(base + q * 512, 512)], wsem)
            w.start()
            wds.append(w)
        wds[-2].wait()
        wds[-1].wait()

    @pl.when(sid == 0)
    def _():
        pltpu.async_copy(tabT_hbm.at[f0], buf0, sem0).wait()

    plsc.subcore_barrier()

    def _feature_body(i, carry):
        fa = f0 + 2 * i
        d1 = pltpu.make_async_copy(tabT_hbm.at[fa + 1], buf1, sem1)

        @pl.when(sid == 0)
        def _():
            d1.start()

        _gather_row(buf0, fa)

        @pl.when(sid == 0)
        def _():
            d1.wait()

        plsc.subcore_barrier()
        more = jnp.logical_and(sid == 0, i < _FPC // 2 - 1)
        d0 = pltpu.make_async_copy(
            tabT_hbm.at[jnp.minimum(fa + 2, _HASH_DIM - 1)], buf0, sem0)

        @pl.when(more)
        def _():
            d0.start()

        _gather_row(buf1, fa + 1)

        @pl.when(more)
        def _():
            d0.wait()

        plsc.subcore_barrier()
        return carry

    lax.fori_loop(0, _FPC // 2, _feature_body, 0)


_gather_call = functools.partial(
    pl.kernel,
    out_type=jax.ShapeDtypeStruct((_HASH_DIM, _TOK), jnp.float32),
    mesh=plsc.VectorSubcoreMesh(
        core_axis_name="c", subcore_axis_name="s",
        num_cores=_NC, num_subcores=_NS),
    scratch_types=[
        pltpu.VMEM((_IDC,), jnp.int32),
        pltpu.VMEM((_IDC,), jnp.int32),
        pltpu.VMEM((_NG, _GBLK), jnp.int32),
        pltpu.VMEM((1024,), jnp.float32),
        pltpu.VMEM_SHARED((_NUM_BUCKETS,), jnp.float32),
        pltpu.VMEM_SHARED((_NUM_BUCKETS,), jnp.float32),
        pltpu.SemaphoreType.DMA,
        pltpu.SemaphoreType.DMA,
        pltpu.SemaphoreType.DMA,
        pltpu.SemaphoreType.DMA,
    ],
)(_sc_gather)


def _mm_body(embT_ref, w_ref, o_ref):
    o_ref[...] = lax.dot_general(
        embT_ref[...], w_ref[...], (((0,), (1,)), ((), ())),
        preferred_element_type=jnp.float32)


def kernel(input_ids, table, W):
    ids = input_ids.astype(jnp.int32)
    bsz, seqlen = ids.shape
    prev = jnp.concatenate(
        [jnp.zeros((bsz, 1), dtype=ids.dtype), ids[:, :-1]], axis=1)
    ids_f = ids.reshape(-1)
    prev_f = prev.reshape(-1)

    embT = _gather_call(ids_f, prev_f, table.T)

    blk = 1024
    out = pl.pallas_call(
        _mm_body,
        grid=(_TOK // blk,),
        in_specs=[
            pl.BlockSpec((_HASH_DIM, blk), lambda i: (0, i)),
            pl.BlockSpec((_MODEL_DIM, _HASH_DIM), lambda i: (0, 0)),
        ],
        out_specs=pl.BlockSpec((blk, _MODEL_DIM), lambda i: (i, 0)),
        out_shape=jax.ShapeDtypeStruct((_TOK, _MODEL_DIM), jnp.float32),
    )(embT, W)
    return out.reshape(bsz, seqlen, _MODEL_DIM)


# speedup vs baseline: 3.5693x; 1.0235x over previous
import functools

import jax
import jax.numpy as jnp
from jax import lax
from jax.experimental import pallas as pl
from jax.experimental.pallas import tpu as pltpu
from jax.experimental.pallas import tpu_sc as plsc

_NUM_BUCKETS = 1000000
_HASH_DIM = 64
_MODEL_DIM = 1024
_MULT = 92821

_NC = 2
_NS = 16
_L = 16

_TOK = 32768
_TPT = _TOK // _NS
_FPC = _HASH_DIM // _NC
_GBLK = 128
_NG = _TPT // _GBLK


_IDC = 256


def _sc_gather(ids_hbm, prev_hbm, tabT_hbm, embT_hbm,
               ids_v, prev_v, h_v, stage_v, buf0, buf1, sem0, sem1, gsem,
               wsem):
    cid = lax.axis_index("c")
    sid = lax.axis_index("s")
    base = sid * _TPT
    f0 = cid * _FPC
    pd = pltpu.make_async_copy(tabT_hbm.at[f0], buf0, sem0)

    @pl.when(sid == 0)
    def _():
        pd.start()

    def _hash_body(c, carry):
        pltpu.sync_copy(ids_hbm.at[pl.ds(base + c * _IDC, _IDC)], ids_v)
        pltpu.sync_copy(prev_hbm.at[pl.ds(base + c * _IDC, _IDC)], prev_v)
        for k in range(_IDC // _L):
            a = prev_v[pl.ds(k * _L, _L)]
            b = ids_v[pl.ds(k * _L, _L)]
            t = a * _MULT + b
            r = lax.rem(t, _NUM_BUCKETS)
            h = jnp.where(r < 0, r + _NUM_BUCKETS, r)
            h_v[c * (_IDC // _GBLK) + k // (_GBLK // _L),
                pl.ds((k % (_GBLK // _L)) * _L, _L)] = h
        return carry

    lax.fori_loop(0, _TPT // _IDC, _hash_body, 0)

    _NST = _TPT // 512

    def _gather_row(buf, f):
        wds = []
        for q in range(_NST):
            st = stage_v.at[pl.ds((q % 2) * 512, 512)]
            if q >= 2:
                wds[q - 2].wait()
            gds = []
            for j in range(4):
                jj = q * 4 + j
                gds.append(pltpu.async_copy(
                    buf.at[h_v.at[jj]],
                    st.at[pl.ds(j * _GBLK, _GBLK)], gsem))
            for d in gds:
                d.wait()
            w = pltpu.make_async_copy(
                st, embT_hbm.at[f, pl.ds(base + q * 512, 512)], wsem)
            w.start()
            wds.append(w)
        wds[-2].wait()
        wds[-1].wait()

    @pl.when(sid == 0)
    def _():
        pd.wait()

    plsc.subcore_barrier()

    def _feature_body(i, carry):
        fa = f0 + 2 * i
        d1 = pltpu.make_async_copy(tabT_hbm.at[fa + 1], buf1, sem1)

        @pl.when(sid == 0)
        def _():
            d1.start()

        _gather_row(buf0, fa)

        @pl.when(sid == 0)
        def _():
            d1.wait()

        plsc.subcore_barrier()
        more = jnp.logical_and(sid == 0, i < _FPC // 2 - 1)
        d0 = pltpu.make_async_copy(
            tabT_hbm.at[jnp.minimum(fa + 2, _HASH_DIM - 1)], buf0, sem0)

        @pl.when(more)
        def _():
            d0.start()

        _gather_row(buf1, fa + 1)

        @pl.when(more)
        def _():
            d0.wait()

        plsc.subcore_barrier()
        return carry

    lax.fori_loop(0, _FPC // 2, _feature_body, 0)


_gather_call = functools.partial(
    pl.kernel,
    out_type=jax.ShapeDtypeStruct((_HASH_DIM, _TOK), jnp.float32),
    mesh=plsc.VectorSubcoreMesh(
        core_axis_name="c", subcore_axis_name="s",
        num_cores=_NC, num_subcores=_NS),
    scratch_types=[
        pltpu.VMEM((_IDC,), jnp.int32),
        pltpu.VMEM((_IDC,), jnp.int32),
        pltpu.VMEM((_NG, _GBLK), jnp.int32),
        pltpu.VMEM((1024,), jnp.float32),
        pltpu.VMEM_SHARED((_NUM_BUCKETS,), jnp.float32),
        pltpu.VMEM_SHARED((_NUM_BUCKETS,), jnp.float32),
        pltpu.SemaphoreType.DMA,
        pltpu.SemaphoreType.DMA,
        pltpu.SemaphoreType.DMA,
        pltpu.SemaphoreType.DMA,
    ],
)(_sc_gather)


def _mm_body(embT_ref, w_ref, o_ref):
    o_ref[...] = lax.dot_general(
        embT_ref[...], w_ref[...], (((0,), (1,)), ((), ())),
        preferred_element_type=jnp.float32)


def kernel(input_ids, table, W):
    ids = input_ids.astype(jnp.int32)
    bsz, seqlen = ids.shape
    prev = jnp.concatenate(
        [jnp.zeros((bsz, 1), dtype=ids.dtype), ids[:, :-1]], axis=1)
    ids_f = ids.reshape(-1)
    prev_f = prev.reshape(-1)

    embT = _gather_call(ids_f, prev_f, table.T)

    blk = 1024
    out = pl.pallas_call(
        _mm_body,
        grid=(_TOK // blk,),
        in_specs=[
            pl.BlockSpec((_HASH_DIM, blk), lambda i: (0, i)),
            pl.BlockSpec((_MODEL_DIM, _HASH_DIM), lambda i: (0, 0)),
        ],
        out_specs=pl.BlockSpec((blk, _MODEL_DIM), lambda i: (i, 0)),
        out_shape=jax.ShapeDtypeStruct((_TOK, _MODEL_DIM), jnp.float32),
    )(embT, W)
    return out.reshape(bsz, seqlen, _MODEL_DIM)
